# Initial kernel scaffold; baseline (speedup 1.0000x reference)
import jax
import jax.numpy as jnp
from jax.experimental import pallas as pl


def kernel(pairs, labels, A, emb, Ws, W1, b1, W2, b2):
    raise NotImplementedError("write your pallas kernel here")


# speedup vs baseline: 1.3190x; 1.3190x over previous
import functools

import jax
import jax.numpy as jnp
from jax import lax
from jax.experimental import pallas as pl
from jax.experimental.pallas import tpu as pltpu
from jax.experimental.pallas import tpu_sc as plsc


def _xw_body(h_ref, w_ref, o_ref):
    o_ref[...] = jnp.dot(
        h_ref[...].astype(jnp.bfloat16), w_ref[...],
        preferred_element_type=jnp.float32).astype(jnp.bfloat16)


def _layer_fused_body(a_ref, x_ref, w_ref, o_ref):
    acc = jnp.dot(a_ref[...].astype(jnp.bfloat16), x_ref[...],
                  preferred_element_type=jnp.float32)
    h = jnp.maximum(acc, 0.0).astype(jnp.bfloat16)
    o_ref[...] = jnp.dot(h, w_ref[...],
                         preferred_element_type=jnp.float32).astype(jnp.bfloat16)


def _layer_final_body(a_ref, x_ref, o_ref):
    acc = jnp.dot(a_ref[...].astype(jnp.bfloat16), x_ref[...],
                  preferred_element_type=jnp.float32)
    o_ref[...] = jnp.maximum(acc, 0.0)


def _mlp_body(se_ref, de_ref, lab_ref, w1_ref, b1_ref, w2_ref, b2_ref, o_ref,
              *, d_model, b_total):
    i = pl.program_id(0)
    w1 = w1_ref[...].astype(jnp.bfloat16)
    h = jnp.dot(se_ref[...].astype(jnp.bfloat16), w1[:d_model],
                preferred_element_type=jnp.float32)
    h += jnp.dot(de_ref[...].astype(jnp.bfloat16), w1[d_model:],
                 preferred_element_type=jnp.float32)
    h = jnp.maximum(h + b1_ref[...], 0.0)
    l = jnp.dot(h, w2_ref[...], preferred_element_type=jnp.float32)
    b2 = b2_ref[...]
    d = (l[:, 1:2] - l[:, 0:1]) + (b2[:, 1:2] - b2[:, 0:1])
    u = jnp.tanh(d * 0.5)
    s = 1.0 - 2.0 * lab_ref[...].astype(jnp.float32)
    nll = jnp.log1p(jnp.exp(s * u))
    part = jnp.sum(nll) * (1.0 / b_total)

    @pl.when(i == 0)
    def _():
        o_ref[...] = jnp.zeros_like(o_ref)

    o_ref[...] += part


_SC_WORKERS = 32
_SC_CHUNK = 128


def _make_sc_gather(n_idx, d_model, dtype):
    per_w = n_idx // _SC_WORKERS
    n_chunks = per_w // _SC_CHUNK
    mesh = plsc.VectorSubcoreMesh(core_axis_name="c", subcore_axis_name="s")

    @functools.partial(
        pl.kernel, mesh=mesh,
        out_type=jax.ShapeDtypeStruct((n_idx, d_model), dtype),
        scratch_types=[
            pltpu.VMEM((_SC_CHUNK,), jnp.int32),
            pltpu.VMEM((_SC_CHUNK, d_model), dtype),
            pltpu.SemaphoreType.DMA,
        ],
    )
    def _gather(table_hbm, idx_hbm, out_hbm, idx_v, rows_v, sem):
        wid = lax.axis_index("s") * 2 + lax.axis_index("c")
        base = wid * per_w
        for c in range(n_chunks):
            off = base + c * _SC_CHUNK
            pltpu.sync_copy(idx_hbm.at[pl.ds(off, _SC_CHUNK)], idx_v)
            pltpu.async_copy(table_hbm.at[idx_v], rows_v, sem).wait()
            pltpu.sync_copy(rows_v, out_hbm.at[pl.ds(off, _SC_CHUNK)])

    return _gather


_ROW_BLK = 200
_PAIR_BLK = 2048


def kernel(pairs, labels, A, emb, Ws, W1, b1, W2, b2):
    n, d = emb.shape
    b_pairs = pairs.shape[0]
    wsb = Ws.astype(jnp.bfloat16)

    x0 = pl.pallas_call(
        _xw_body,
        out_shape=jax.ShapeDtypeStruct((n, d), jnp.bfloat16),
    )(emb, wsb[0])

    grid = (n // _ROW_BLK,)
    a_spec = pl.BlockSpec((_ROW_BLK, n), lambda i: (i, 0))
    x_spec = pl.BlockSpec((n, d), lambda i: (0, 0))
    o_spec = pl.BlockSpec((_ROW_BLK, d), lambda i: (i, 0))
    params = pltpu.CompilerParams(vmem_limit_bytes=100 * 1024 * 1024)

    x1 = pl.pallas_call(
        _layer_fused_body,
        grid=grid,
        in_specs=[a_spec, x_spec, pl.BlockSpec((d, d), lambda i: (0, 0))],
        out_specs=o_spec,
        out_shape=jax.ShapeDtypeStruct((n, d), jnp.bfloat16),
        compiler_params=params,
    )(A, x0, wsb[1])

    h2 = pl.pallas_call(
        _layer_final_body,
        grid=grid,
        in_specs=[a_spec, x_spec],
        out_specs=o_spec,
        out_shape=jax.ShapeDtypeStruct((n, d), jnp.float32),
        compiler_params=params,
    )(A, x1)

    idx = jnp.concatenate([pairs[:, 0], pairs[:, 1]]).astype(jnp.int32)
    gathered = _make_sc_gather(2 * b_pairs, d, jnp.float32)(h2, idx)

    n_blk = b_pairs // _PAIR_BLK
    loss = pl.pallas_call(
        functools.partial(_mlp_body, d_model=d, b_total=float(b_pairs)),
        grid=(n_blk,),
        in_specs=[
            pl.BlockSpec((_PAIR_BLK, d), lambda i: (i, 0)),
            pl.BlockSpec((_PAIR_BLK, d), lambda i: (n_blk + i, 0)),
            pl.BlockSpec((_PAIR_BLK, 1), lambda i: (i, 0)),
            pl.BlockSpec((2 * d, d), lambda i: (0, 0)),
            pl.BlockSpec((1, d), lambda i: (0, 0)),
            pl.BlockSpec((d, 2), lambda i: (0, 0)),
            pl.BlockSpec((1, 2), lambda i: (0, 0)),
        ],
        out_specs=pl.BlockSpec((1, 1), lambda i: (0, 0)),
        out_shape=jax.ShapeDtypeStruct((1, 1), jnp.float32),
    )(gathered, gathered, labels.astype(jnp.int32).reshape(b_pairs, 1),
      W1, b1.reshape(1, d), W2, b2.reshape(1, 2))

    return loss.reshape(())


# speedup vs baseline: 1.3300x; 1.0083x over previous
import functools

import jax
import jax.numpy as jnp
from jax import lax
from jax.experimental import pallas as pl
from jax.experimental.pallas import tpu as pltpu
from jax.experimental.pallas import tpu_sc as plsc


def _xw_body(h_ref, w_ref, o_ref):
    o_ref[...] = jnp.dot(
        h_ref[...].astype(jnp.bfloat16), w_ref[...],
        preferred_element_type=jnp.float32).astype(jnp.bfloat16)


def _layer_fused_body(a_ref, x_ref, w_ref, o_ref):
    acc = jnp.dot(a_ref[...].astype(jnp.bfloat16), x_ref[...],
                  preferred_element_type=jnp.float32)
    h = jnp.maximum(acc, 0.0).astype(jnp.bfloat16)
    o_ref[...] = jnp.dot(h, w_ref[...],
                         preferred_element_type=jnp.float32).astype(jnp.bfloat16)


def _layer_final_body(a_ref, x_ref, o_ref):
    acc = jnp.dot(a_ref[...].astype(jnp.bfloat16), x_ref[...],
                  preferred_element_type=jnp.float32)
    o_ref[...] = jnp.maximum(acc, 0.0)


def _mlp_body(se_ref, de_ref, lab_ref, w1_ref, b1_ref, w2_ref, b2_ref, o_ref,
              *, d_model, b_total):
    i = pl.program_id(0)
    w1 = w1_ref[...].astype(jnp.bfloat16)
    h = jnp.dot(se_ref[...].astype(jnp.bfloat16), w1[:d_model],
                preferred_element_type=jnp.float32)
    h += jnp.dot(de_ref[...].astype(jnp.bfloat16), w1[d_model:],
                 preferred_element_type=jnp.float32)
    h = jnp.maximum(h + b1_ref[...], 0.0)
    l = jnp.dot(h, w2_ref[...], preferred_element_type=jnp.float32)
    b2 = b2_ref[...]
    d = (l[:, 1:2] - l[:, 0:1]) + (b2[:, 1:2] - b2[:, 0:1])
    u = jnp.tanh(d * 0.5)
    s = 1.0 - 2.0 * lab_ref[...].astype(jnp.float32)
    nll = jnp.log1p(jnp.exp(s * u))
    part = jnp.sum(nll) * (1.0 / b_total)

    @pl.when(i == 0)
    def _():
        o_ref[...] = jnp.zeros_like(o_ref)

    o_ref[...] += part


_SC_WORKERS = 32
_SC_CHUNK = 128


def _make_sc_gather(n_idx, d_model, dtype):
    per_w = n_idx // _SC_WORKERS
    n_chunks = per_w // _SC_CHUNK
    mesh = plsc.VectorSubcoreMesh(core_axis_name="c", subcore_axis_name="s")

    @functools.partial(
        pl.kernel, mesh=mesh,
        out_type=jax.ShapeDtypeStruct((n_idx, d_model), dtype),
        scratch_types=[
            pltpu.VMEM((_SC_CHUNK,), jnp.int32),
            pltpu.VMEM((_SC_CHUNK, d_model), dtype),
            pltpu.SemaphoreType.DMA,
        ],
    )
    def _gather(table_hbm, idx_hbm, out_hbm, idx_v, rows_v, sem):
        wid = lax.axis_index("s") * 2 + lax.axis_index("c")
        base = wid * per_w
        for c in range(n_chunks):
            off = base + c * _SC_CHUNK
            pltpu.sync_copy(idx_hbm.at[pl.ds(off, _SC_CHUNK)], idx_v)
            pltpu.async_copy(table_hbm.at[idx_v], rows_v, sem).wait()
            pltpu.sync_copy(rows_v, out_hbm.at[pl.ds(off, _SC_CHUNK)])

    return _gather


_ROW_BLK = 400
_PAIR_BLK = 2048


def kernel(pairs, labels, A, emb, Ws, W1, b1, W2, b2):
    n, d = emb.shape
    b_pairs = pairs.shape[0]
    wsb = Ws.astype(jnp.bfloat16)

    x0 = pl.pallas_call(
        _xw_body,
        out_shape=jax.ShapeDtypeStruct((n, d), jnp.bfloat16),
    )(emb, wsb[0])

    grid = (n // _ROW_BLK,)
    a_spec = pl.BlockSpec((_ROW_BLK, n), lambda i: (i, 0))
    x_spec = pl.BlockSpec((n, d), lambda i: (0, 0))
    o_spec = pl.BlockSpec((_ROW_BLK, d), lambda i: (i, 0))
    params = pltpu.CompilerParams(
        vmem_limit_bytes=100 * 1024 * 1024,
        dimension_semantics=("parallel",),
    )

    x1 = pl.pallas_call(
        _layer_fused_body,
        grid=grid,
        in_specs=[a_spec, x_spec, pl.BlockSpec((d, d), lambda i: (0, 0))],
        out_specs=o_spec,
        out_shape=jax.ShapeDtypeStruct((n, d), jnp.bfloat16),
        compiler_params=params,
    )(A, x0, wsb[1])

    h2 = pl.pallas_call(
        _layer_final_body,
        grid=grid,
        in_specs=[a_spec, x_spec],
        out_specs=o_spec,
        out_shape=jax.ShapeDtypeStruct((n, d), jnp.float32),
        compiler_params=params,
    )(A, x1)

    idx = jnp.concatenate([pairs[:, 0], pairs[:, 1]]).astype(jnp.int32)
    gathered = _make_sc_gather(2 * b_pairs, d, jnp.float32)(h2, idx)

    n_blk = b_pairs // _PAIR_BLK
    loss = pl.pallas_call(
        functools.partial(_mlp_body, d_model=d, b_total=float(b_pairs)),
        grid=(n_blk,),
        in_specs=[
            pl.BlockSpec((_PAIR_BLK, d), lambda i: (i, 0)),
            pl.BlockSpec((_PAIR_BLK, d), lambda i: (n_blk + i, 0)),
            pl.BlockSpec((_PAIR_BLK, 1), lambda i: (i, 0)),
            pl.BlockSpec((2 * d, d), lambda i: (0, 0)),
            pl.BlockSpec((1, d), lambda i: (0, 0)),
            pl.BlockSpec((d, 2), lambda i: (0, 0)),
            pl.BlockSpec((1, 2), lambda i: (0, 0)),
        ],
        out_specs=pl.BlockSpec((1, 1), lambda i: (0, 0)),
        out_shape=jax.ShapeDtypeStruct((1, 1), jnp.float32),
    )(gathered, gathered, labels.astype(jnp.int32).reshape(b_pairs, 1),
      W1, b1.reshape(1, d), W2, b2.reshape(1, 2))

    return loss.reshape(())
